# Initial kernel scaffold; baseline (speedup 1.0000x reference)
import jax
import jax.numpy as jnp
from jax.experimental import pallas as pl


def kernel(Rij, senders, receivers, species, n_node, params):
    raise NotImplementedError("write your pallas kernel here")


# speedup vs baseline: 1.7030x; 1.7030x over previous
import functools
import numpy as np
import jax
import jax.numpy as jnp
from jax import lax
from jax.experimental import pallas as pl
from jax.experimental.pallas import tpu as pltpu
from jax.experimental.pallas import tpu_sc as plsc

N = 10000
E = 320000
NUM_SPECIES = 5
CUTOFF = 5.0
AVG_NEIGH = 32.0
NB = 8
FDIM = 128
NLAYERS = 4
SCAL = 32
NP1 = 8
NP2 = 4

EPAD = 327680
BE = 2560
BN = 2000
C = 256
CH_PER_TILE = EPAD // (32 * C)
NPAD = 10240
ROWS_T = NPAD // 16

def _consts():
    p4480 = np.zeros((44, 80), np.float32)
    for k in range(SCAL):
        p4480[k, k] = 1.0
    for p in range(NP1):
        for c in range(3):
            p4480[SCAL + p, SCAL + 3 * p + c] = 1.0
    for p in range(NP2):
        for c in range(5):
            p4480[SCAL + NP1 + p, SCAL + 24 + 5 * p + c] = 1.0
    q980 = np.zeros((9, 80), np.float32)
    q980[0, :SCAL] = 1.0
    for p in range(NP1):
        for c in range(3):
            q980[1 + c, SCAL + 3 * p + c] = 1.0
    for p in range(NP2):
        for c in range(5):
            q980[4 + c, SCAL + 24 + 5 * p + c] = 1.0
    rsum = np.zeros((80, 12), np.float32)
    for p in range(NP1):
        for c in range(3):
            rsum[SCAL + 3 * p + c, p] = 1.0
    for p in range(NP2):
        for c in range(5):
            rsum[SCAL + 24 + 5 * p + c, NP1 + p] = 1.0
    return p4480, q980, rsum

_P4480, _Q980, _RSUM = _consts()


def _silu(x):
    return x * jax.nn.sigmoid(x)


def _edge_w_body(rij_ref, wr1_ref, wr2_ref, p_ref, q_ref, out_ref):
    r = rij_ref[...] * (1.0 / CUTOFF)
    l2 = jnp.sum(r * r, axis=1, keepdims=True)
    lengths = jnp.sqrt(l2)
    iszero = lengths == 0.0
    safe = jnp.where(iszero, 1.0, lengths)
    inv_safe = 1.0 / safe
    u = r * inv_safe
    x = u[:, 0:1]
    y = u[:, 1:2]
    z = u[:, 2:3]
    s3 = np.sqrt(3.0).astype(np.float32)
    s15 = np.sqrt(15.0).astype(np.float32)
    terms = [
        jnp.ones_like(x),
        s3 * x, s3 * y, s3 * z,
        s15 * x * y,
        s15 * y * z,
        np.float32(np.sqrt(5.0) / 2.0) * (3.0 * z * z - 1.0),
        s15 * x * z,
        np.float32(np.sqrt(15.0) / 2.0) * (x * x - y * y),
    ]
    yq = jnp.zeros((rij_ref.shape[0], 80), jnp.float32)
    for k, t in enumerate(terms):
        yq = yq + t * q_ref[k:k + 1, :]
    nfreq = ((lax.broadcasted_iota(jnp.int32, (1, NB), 1) + 1)
             .astype(jnp.float32) * np.float32(np.pi))
    rb = np.float32(np.sqrt(2.0)) * jnp.sin(lengths * nfreq) * inv_safe
    rb = jnp.where(iszero, 0.0, rb)
    rw = _silu(rb @ wr1_ref[0]) @ wr2_ref[0]
    out_ref[0] = (rw @ p_ref[...]) * yq


def _edge_weights(rij_pad, wr1s, wr2s):
    nblk = EPAD // BE
    return pl.pallas_call(
        _edge_w_body,
        grid=(NLAYERS, nblk),
        in_specs=[
            pl.BlockSpec((BE, 3), lambda l, i: (i, 0)),
            pl.BlockSpec((1, NB, 64), lambda l, i: (l, 0, 0)),
            pl.BlockSpec((1, 64, 44), lambda l, i: (l, 0, 0)),
            pl.BlockSpec((44, 80), lambda l, i: (0, 0)),
            pl.BlockSpec((9, 80), lambda l, i: (0, 0)),
        ],
        out_specs=pl.BlockSpec((1, BE, 80), lambda l, i: (l, i, 0)),
        out_shape=jax.ShapeDtypeStruct((NLAYERS, EPAD, 80), jnp.float32),
    )(rij_pad, wr1s, wr2s, jnp.asarray(_P4480), jnp.asarray(_Q980))


def _node0_body(spec_ref, emb_ref, ws_ref, wp_ref, wd_ref, wself_ref, p_ref,
                soh_ref, pexp_ref, skip_ref):
    soh = (spec_ref[...] == lax.broadcasted_iota(jnp.int32, (1, NUM_SPECIES), 1))
    soh = soh.astype(jnp.float32)
    soh_ref[...] = soh
    scal = soh @ emb_ref[...]
    pexp = ((scal @ ws_ref[...]) @ p_ref[0:SCAL, :]
            + (scal @ wp_ref[...]) @ p_ref[SCAL:SCAL + NP1, :]
            + (scal @ wd_ref[...]) @ p_ref[SCAL + NP1:44, :])
    pexp_ref[...] = pexp
    skip = jnp.zeros((spec_ref.shape[0], SCAL), jnp.float32)
    for s in range(NUM_SPECIES):
        skip = skip + soh[:, s:s + 1] * (scal @ wself_ref[s])
    skip_ref[...] = skip


def _node0(spec2, emb, ws, wp, wd, wself):
    nblk = N // BN
    return pl.pallas_call(
        _node0_body,
        grid=(nblk,),
        in_specs=[
            pl.BlockSpec((BN, 1), lambda i: (i, 0)),
            pl.BlockSpec((NUM_SPECIES, FDIM), lambda i: (0, 0)),
            pl.BlockSpec((FDIM, SCAL), lambda i: (0, 0)),
            pl.BlockSpec((FDIM, NP1), lambda i: (0, 0)),
            pl.BlockSpec((FDIM, NP2), lambda i: (0, 0)),
            pl.BlockSpec((NUM_SPECIES, FDIM, SCAL), lambda i: (0, 0, 0)),
            pl.BlockSpec((44, 80), lambda i: (0, 0)),
        ],
        out_specs=[
            pl.BlockSpec((BN, NUM_SPECIES), lambda i: (i, 0)),
            pl.BlockSpec((BN, 80), lambda i: (i, 0)),
            pl.BlockSpec((BN, SCAL), lambda i: (i, 0)),
        ],
        out_shape=[
            jax.ShapeDtypeStruct((N, NUM_SPECIES), jnp.float32),
            jax.ShapeDtypeStruct((N, 80), jnp.float32),
            jax.ShapeDtypeStruct((N, SCAL), jnp.float32),
        ],
    )(spec2, emb, ws, wp, wd, wself, jnp.asarray(_P4480))


def _nodeab_body(l_ref, acc_ref, skip_ref, soh_ref, rsum_ref, winv_ref,
                 ws_ref, wp_ref, wd_ref, wself_ref, p_ref,
                 pexp_ref, skipn_ref):
    del l_ref
    acc = (acc_ref[0] + acc_ref[1]) * (1.0 / AVG_NEIGH)
    scal = _silu(acc[:, 0:SCAL] + skip_ref[...])
    inv = (acc * acc) @ rsum_ref[...]
    scal = scal + inv @ winv_ref[0]
    pexp = ((scal @ ws_ref[0]) @ p_ref[0:SCAL, :]
            + (scal @ wp_ref[0]) @ p_ref[SCAL:SCAL + NP1, :]
            + (scal @ wd_ref[0]) @ p_ref[SCAL + NP1:44, :])
    pexp_ref[...] = pexp
    soh = soh_ref[...]
    skip = jnp.zeros((acc_ref.shape[1], SCAL), jnp.float32)
    for s in range(NUM_SPECIES):
        skip = skip + soh[:, s:s + 1] * (scal @ wself_ref[0, s])
    skipn_ref[...] = skip


def _nodeab(l, acc2, skip, soh, winvS, wsS, wpS, wdS, wselfS):
    nblk = N // BN
    grid_spec = pltpu.PrefetchScalarGridSpec(
        num_scalar_prefetch=1,
        grid=(nblk,),
        in_specs=[
            pl.BlockSpec((2, BN, 80), lambda i, s: (0, i, 0)),
            pl.BlockSpec((BN, SCAL), lambda i, s: (i, 0)),
            pl.BlockSpec((BN, NUM_SPECIES), lambda i, s: (i, 0)),
            pl.BlockSpec((80, 12), lambda i, s: (0, 0)),
            pl.BlockSpec((1, 12, SCAL), lambda i, s: (s[0], 0, 0)),
            pl.BlockSpec((1, SCAL, SCAL), lambda i, s: (s[0], 0, 0)),
            pl.BlockSpec((1, SCAL, NP1), lambda i, s: (s[0], 0, 0)),
            pl.BlockSpec((1, SCAL, NP2), lambda i, s: (s[0], 0, 0)),
            pl.BlockSpec((1, NUM_SPECIES, SCAL, SCAL),
                         lambda i, s: (s[0], 0, 0, 0)),
            pl.BlockSpec((44, 80), lambda i, s: (0, 0)),
        ],
        out_specs=[
            pl.BlockSpec((BN, 80), lambda i, s: (i, 0)),
            pl.BlockSpec((BN, SCAL), lambda i, s: (i, 0)),
        ],
    )
    return pl.pallas_call(
        _nodeab_body,
        grid_spec=grid_spec,
        out_shape=[
            jax.ShapeDtypeStruct((N, 80), jnp.float32),
            jax.ShapeDtypeStruct((N, SCAL), jnp.float32),
        ],
    )(jnp.full((1,), l, jnp.int32), acc2, skip, soh, jnp.asarray(_RSUM),
      winvS, wsS, wpS, wdS, wselfS, jnp.asarray(_P4480))


def _final_body(acc_ref, skip_ref, rsum_ref, winv_ref, w1_ref, w2_ref, out_ref):
    i = pl.program_id(0)
    acc = (acc_ref[0] + acc_ref[1]) * (1.0 / AVG_NEIGH)
    scal = _silu(acc[:, 0:SCAL] + skip_ref[...])
    inv = (acc * acc) @ rsum_ref[...]
    scal = scal + inv @ winv_ref[...]
    e = (scal @ w1_ref[...]) @ w2_ref[...]
    t = jnp.sum(e[:, 0:1], axis=0, keepdims=True)

    @pl.when(i == 0)
    def _():
        out_ref[...] = jnp.zeros_like(out_ref)

    out_ref[...] += t


def _final(acc2, skip, winv, w1, w2):
    nblk = N // BN
    return pl.pallas_call(
        _final_body,
        grid=(nblk,),
        in_specs=[
            pl.BlockSpec((2, BN, 80), lambda i: (0, i, 0)),
            pl.BlockSpec((BN, SCAL), lambda i: (i, 0)),
            pl.BlockSpec((80, 12), lambda i: (0, 0)),
            pl.BlockSpec((12, SCAL), lambda i: (0, 0)),
            pl.BlockSpec((SCAL, 16), lambda i: (0, 0)),
            pl.BlockSpec((16, 2), lambda i: (0, 0)),
        ],
        out_specs=pl.BlockSpec((1, 1), lambda i: (0, 0)),
        out_shape=jax.ShapeDtypeStruct((1, 1), jnp.float32),
    )(acc2, skip, jnp.asarray(_RSUM), winv, w1, w2)


@functools.cache
def _get_sc_edge_pass():
    mesh = plsc.VectorSubcoreMesh(core_axis_name="c", subcore_axis_name="s",
                                  num_cores=2, num_subcores=16)
    return functools.partial(
        pl.kernel,
        out_type=jax.ShapeDtypeStruct((2 * NPAD, 80), jnp.float32),
        mesh=mesh,
        scratch_types=[
            pltpu.VMEM((C,), jnp.int32),
            [pltpu.VMEM((128,), jnp.int32)] * (C // 128),
            pltpu.VMEM((C, 80), jnp.float32),
            pltpu.VMEM((C, 80), jnp.float32),
            pltpu.VMEM_SHARED((NPAD, 80), jnp.float32),
            pltpu.VMEM((16,), jnp.int32),
            pltpu.VMEM((C,), jnp.int32),
            pltpu.SemaphoreType.DMA,
        ],
        compiler_params=pltpu.CompilerParams(use_tc_tiling_on_sc=False),
    )(_sc_edge_body)


def _sc_edge_body(pexp_hbm, w_hbm, snd_hbm, rcv_hbm, lsel_hbm, out_hbm,
                  sidx, ridx, gbuf, wbuf, acc, lvm, widx, sem):
    cid = lax.axis_index("c")
    sid = lax.axis_index("s")
    tid = cid * 16 + sid
    pltpu.sync_copy(lsel_hbm, lvm)

    def _zrow(i, _):
        for j in range(5):
            gbuf[i, pl.ds(16 * j, 16)] = jnp.zeros((16,), jnp.float32)
        return 0
    lax.fori_loop(0, C, _zrow, 0)
    r0 = sid * ROWS_T
    nfull, rem = divmod(ROWS_T, C)
    for t in range(nfull):
        pltpu.sync_copy(gbuf, acc.at[pl.ds(r0 + t * C, C)])
    if rem:
        pltpu.sync_copy(gbuf.at[pl.ds(0, rem)],
                        acc.at[pl.ds(r0 + nfull * C, rem)])
    plsc.subcore_barrier()

    base_e = tid * (CH_PER_TILE * C)

    def _chunk(k, _):
        b = base_e + k * C
        pltpu.sync_copy(snd_hbm.at[pl.ds(b, C)], sidx)
        for j in range(C // 128):
            pltpu.sync_copy(rcv_hbm.at[pl.ds(b + 128 * j, 128)], ridx[j])
        lv = lvm[...] + b
        ramp = lax.iota(jnp.int32, 16)
        for j in range(C // 16):
            widx[pl.ds(16 * j, 16)] = lv + (ramp + 16 * j)
        pltpu.async_copy(pexp_hbm.at[sidx], gbuf, sem).wait()
        pltpu.async_copy(w_hbm.at[widx], wbuf, sem).wait()

        def _mrow(i, _):
            for j in range(5):
                sl = pl.ds(16 * j, 16)
                gbuf[i, sl] = gbuf[i, sl] * wbuf[i, sl]
            return 0
        lax.fori_loop(0, C, _mrow, 0)
        for j in range(C // 128):
            pltpu.sync_copy(gbuf.at[pl.ds(128 * j, 128)],
                            acc.at[ridx[j]], add=True)
        return 0
    lax.fori_loop(0, CH_PER_TILE, _chunk, 0)
    plsc.subcore_barrier()
    pltpu.sync_copy(acc.at[pl.ds(r0, ROWS_T)],
                    out_hbm.at[pl.ds(cid * NPAD + r0, ROWS_T)])


def kernel(Rij, senders, receivers, species, n_node, params):
    del n_node
    rij_pad = jnp.pad(Rij, ((0, EPAD - E), (0, 0)))
    snd = jnp.pad(senders.astype(jnp.int32), (0, EPAD - E))
    rcv = jnp.pad(receivers.astype(jnp.int32), (0, EPAD - E))
    spec2 = species.astype(jnp.int32).reshape(N, 1)

    wr1s = jnp.stack([params['L%d_Wr1' % l] for l in range(NLAYERS)])
    wr2s = jnp.stack([params['L%d_Wr2' % l] for l in range(NLAYERS)])
    wall = _edge_weights(rij_pad, wr1s, wr2s)
    wall2 = wall.reshape(NLAYERS * EPAD, 80)

    soh, pexp, skip = _node0(
        spec2, params['embed'], params['L0_Ws'], params['L0_Wp'],
        params['L0_Wd'], params['L0_Wself'])

    winvS = jnp.stack([params['L%d_Winv' % l] for l in range(NLAYERS)])
    nxt = [1, 2, 3, 3]
    wsS = jnp.stack([params['L%d_Ws' % l] for l in nxt])
    wpS = jnp.stack([params['L%d_Wp' % l] for l in nxt])
    wdS = jnp.stack([params['L%d_Wd' % l] for l in nxt])
    wselfS = jnp.stack([params['L%d_Wself' % l] for l in nxt])

    sc_call = _get_sc_edge_pass()

    def body(l, carry):
        pexp_l, skip_l, _, _ = carry
        lsel = jnp.full((16,), l * EPAD, jnp.int32)
        accf = sc_call(pexp_l, wall2, snd, rcv, lsel)
        pexp_n, skip_n = _nodeab(l, accf.reshape(2, NPAD, 80), skip_l, soh,
                                 winvS, wsS, wpS, wdS, wselfS)
        return (pexp_n, skip_n, skip_l, accf)

    init = (pexp, skip, skip, jnp.zeros((2 * NPAD, 80), jnp.float32))
    _, _, skip3, acc3f = lax.fori_loop(0, NLAYERS, body, init)
    out = _final(acc3f.reshape(2, NPAD, 80), skip3, params['L3_Winv'],
                 params['Wout1'], params['Wout2'])
    return out.reshape(-1)
